# baseline (speedup 1.0000x reference)
import functools

import jax
import jax.numpy as jnp
from jax import lax
from jax.scipy.linalg import block_diag
from jax.experimental import pallas as pl
from jax.experimental.pallas import tpu as pltpu
from jax.experimental.pallas import tpu_sc as plsc

B = 16384
DU = 32
DI = 64
NH = 64
GU = 128 // DU
GI = 128 // DI

_info = plsc.get_sparse_core_info()
NC = _info.num_cores
NS = _info.num_subcores
NW = NC * NS
BPW = B // NW
CHUNK = 128
NCH = BPW // CHUNK
L = 16

_mesh = plsc.VectorSubcoreMesh(core_axis_name="c", subcore_axis_name="s")


@functools.partial(
    pl.kernel,
    mesh=_mesh,
    out_type=[
        jax.ShapeDtypeStruct((B, 128), jnp.float32),
        jax.ShapeDtypeStruct((B, 128), jnp.float32),
    ],
    scratch_types=[
        pltpu.VMEM((NCH, CHUNK), jnp.int32),
        pltpu.VMEM((NCH, CHUNK), jnp.int32),
        pltpu.VMEM((NCH, CHUNK), jnp.int32),
        pltpu.VMEM((NCH, CHUNK), jnp.int32),
        pltpu.VMEM((2, CHUNK, 128), jnp.float32),
        pltpu.VMEM((2, CHUNK, 128), jnp.float32),
        pltpu.SemaphoreType.DMA,
        pltpu.SemaphoreType.DMA,
        pltpu.SemaphoreType.DMA,
        pltpu.SemaphoreType.DMA,
    ],
)
def _sc_gather(idx_u_hbm, idx_i_hbm, emb_u_hbm, emb_i_hbm,
               hu_hbm, hi_hbm,
               idx_u_v, idx_i_v, g_u_v, g_i_v, buf_u, buf_i,
               sem_gu, sem_gi, sem_wu, sem_wi):
    wid = lax.axis_index("s") * NC + lax.axis_index("c")
    base = wid * BPW
    pltpu.sync_copy(idx_u_hbm.at[wid], idx_u_v)
    pltpu.sync_copy(idx_i_hbm.at[wid], idx_i_v)
    for j in range(NCH):
        for k in range(CHUNK // L):
            sl = pl.ds(k * L, L)
            g_u_v[j, sl] = lax.shift_right_logical(idx_u_v[j, sl], 2)
            g_i_v[j, sl] = lax.shift_right_logical(idx_i_v[j, sl], 1)
    wb_u = [None] * NCH
    wb_i = [None] * NCH
    for j in range(NCH):
        b = j % 2
        if j >= 2:
            wb_u[j - 2].wait()
            wb_i[j - 2].wait()
        gu = pltpu.async_copy(emb_u_hbm.at[g_u_v.at[j]], buf_u.at[b], sem_gu)
        gi = pltpu.async_copy(emb_i_hbm.at[g_i_v.at[j]], buf_i.at[b], sem_gi)
        gu.wait()
        gi.wait()
        dst = pl.ds(base + j * CHUNK, CHUNK)
        wb_u[j] = pltpu.async_copy(buf_u.at[b], hu_hbm.at[dst], sem_wu)
        wb_i[j] = pltpu.async_copy(buf_i.at[b], hi_hbm.at[dst], sem_wi)
    for j in range(NCH - 2, NCH):
        wb_u[j].wait()
        wb_i[j].wait()


def _tc_proj(hu_ref, hi_ref, idxu_ref, idxi_ref,
             wu_ref, bu_ref, wi_ref, bi_ref, ou_ref, oi_ref):
    fu = jnp.dot(hu_ref[...], wu_ref[...], preferred_element_type=jnp.float32)
    su = idxu_ref[...] & (GU - 1)
    o = jnp.where(su == 0, fu[:, 0:64],
                  jnp.where(su == 1, fu[:, 64:128],
                            jnp.where(su == 2, fu[:, 128:192], fu[:, 192:256])))
    ou_ref[...] = jnp.maximum(o + bu_ref[...], 0.0)
    fi = jnp.dot(hi_ref[...], wi_ref[...], preferred_element_type=jnp.float32)
    si = idxi_ref[...] & (GI - 1)
    o2 = jnp.where(si == 0, fi[:, 0:64], fi[:, 64:128])
    oi_ref[...] = jnp.maximum(o2 + bi_ref[...], 0.0)


RB = 2048


def kernel(idx_user, idx_item, emb_user, emb_item, W_user, b_user, W_item, b_item):
    idx_u = idx_user.astype(jnp.int32)
    idx_i = idx_item.astype(jnp.int32)
    idx_u3 = idx_u.reshape(NW, NCH, CHUNK)
    idx_i3 = idx_i.reshape(NW, NCH, CHUNK)
    emb_u2 = emb_user.reshape(V_U_GROUPS, 128)
    emb_i2 = emb_item.reshape(V_I_GROUPS, 128)
    hu_raw, hi_raw = _sc_gather(idx_u3, idx_i3, emb_u2, emb_i2)

    wu_cat = block_diag(*([W_user] * GU))
    wi_cat = block_diag(*([W_item] * GI))

    grid = (B // RB,)
    ou, oi = pl.pallas_call(
        _tc_proj,
        grid=grid,
        in_specs=[
            pl.BlockSpec((RB, 128), lambda i: (i, 0)),
            pl.BlockSpec((RB, 128), lambda i: (i, 0)),
            pl.BlockSpec((RB, 1), lambda i: (i, 0)),
            pl.BlockSpec((RB, 1), lambda i: (i, 0)),
            pl.BlockSpec((128, GU * NH), lambda i: (0, 0)),
            pl.BlockSpec((1, NH), lambda i: (0, 0)),
            pl.BlockSpec((128, GI * NH), lambda i: (0, 0)),
            pl.BlockSpec((1, NH), lambda i: (0, 0)),
        ],
        out_specs=[
            pl.BlockSpec((RB, NH), lambda i: (i, 0)),
            pl.BlockSpec((RB, NH), lambda i: (i, 0)),
        ],
        out_shape=[
            jax.ShapeDtypeStruct((B, NH), jnp.float32),
            jax.ShapeDtypeStruct((B, NH), jnp.float32),
        ],
    )(hu_raw, hi_raw, idx_u.reshape(B, 1), idx_i.reshape(B, 1),
      wu_cat, b_user.reshape(1, NH), wi_cat, b_item.reshape(1, NH))
    return (ou, oi)


V_U_GROUPS = (1000000 * DU) // 128
V_I_GROUPS = (100000 * DI) // 128


# speedup vs baseline: 1.0454x; 1.0454x over previous
import functools

import jax
import jax.numpy as jnp
from jax import lax
from jax.experimental import pallas as pl
from jax.experimental.pallas import tpu as pltpu
from jax.experimental.pallas import tpu_sc as plsc

B = 16384
DU = 32
DI = 64
NH = 64
VU = 1000000
VI = 100000

_info = plsc.get_sparse_core_info()
NC = _info.num_cores
NS = _info.num_subcores
NW = NC * NS
BPW = B // NW
CHUNK = 128
NCH = BPW // CHUNK
L = 16

VB = 8192

_mesh = plsc.VectorSubcoreMesh(core_axis_name="c", subcore_axis_name="s")


def _proj_kernel(t_ref, w_ref, b_ref, o_ref):
    x = t_ref[...]
    y = lax.dot_general(x, w_ref[...], (((0,), (0,)), ((), ())),
                        preferred_element_type=jnp.float32)
    y = jnp.maximum(y + b_ref[...], 0.0)
    y3 = y.reshape(VB // 2, 2, NH)
    o_ref[...] = jnp.concatenate([y3[:, 0, :], y3[:, 1, :]], axis=1)


def _project(table_t, w, b, v):
    d = table_t.shape[0]
    grid = (pl.cdiv(v, VB),)
    return pl.pallas_call(
        _proj_kernel,
        grid=grid,
        in_specs=[
            pl.BlockSpec((d, VB), lambda i: (0, i)),
            pl.BlockSpec((d, NH), lambda i: (0, 0)),
            pl.BlockSpec((1, NH), lambda i: (0, 0)),
        ],
        out_specs=pl.BlockSpec((VB // 2, 2 * NH), lambda i: (i, 0)),
        out_shape=jax.ShapeDtypeStruct((v // 2, 2 * NH), jnp.float32),
    )(table_t, w, b.reshape(1, NH))


@functools.partial(
    pl.kernel,
    mesh=_mesh,
    out_type=[
        jax.ShapeDtypeStruct((B, 128), jnp.float32),
        jax.ShapeDtypeStruct((B, 128), jnp.float32),
    ],
    scratch_types=[
        pltpu.VMEM((NCH, CHUNK), jnp.int32),
        pltpu.VMEM((NCH, CHUNK), jnp.int32),
        pltpu.VMEM((NCH, CHUNK), jnp.int32),
        pltpu.VMEM((NCH, CHUNK), jnp.int32),
        pltpu.VMEM((2, CHUNK, 128), jnp.float32),
        pltpu.VMEM((2, CHUNK, 128), jnp.float32),
        pltpu.SemaphoreType.DMA,
        pltpu.SemaphoreType.DMA,
        pltpu.SemaphoreType.DMA,
        pltpu.SemaphoreType.DMA,
    ],
)
def _sc_gather(idx_u_hbm, idx_i_hbm, pu_hbm, pi_hbm,
               hu_hbm, hi_hbm,
               idx_u_v, idx_i_v, g_u_v, g_i_v, buf_u, buf_i,
               sem_gu, sem_gi, sem_wu, sem_wi):
    wid = lax.axis_index("s") * NC + lax.axis_index("c")
    base = wid * BPW
    pltpu.sync_copy(idx_u_hbm.at[wid], idx_u_v)
    pltpu.sync_copy(idx_i_hbm.at[wid], idx_i_v)
    for j in range(NCH):
        for k in range(CHUNK // L):
            sl = pl.ds(k * L, L)
            g_u_v[j, sl] = lax.shift_right_logical(idx_u_v[j, sl], 1)
            g_i_v[j, sl] = lax.shift_right_logical(idx_i_v[j, sl], 1)
    wb_u = [None] * NCH
    wb_i = [None] * NCH
    for j in range(NCH):
        bsel = j % 2
        if j >= 2:
            wb_u[j - 2].wait()
            wb_i[j - 2].wait()
        gu = pltpu.async_copy(pu_hbm.at[g_u_v.at[j]], buf_u.at[bsel], sem_gu)
        gi = pltpu.async_copy(pi_hbm.at[g_i_v.at[j]], buf_i.at[bsel], sem_gi)
        gu.wait()
        gi.wait()
        dst = pl.ds(base + j * CHUNK, CHUNK)
        wb_u[j] = pltpu.async_copy(buf_u.at[bsel], hu_hbm.at[dst], sem_wu)
        wb_i[j] = pltpu.async_copy(buf_i.at[bsel], hi_hbm.at[dst], sem_wi)
    for j in range(NCH - 2, NCH):
        wb_u[j].wait()
        wb_i[j].wait()


def _select_kernel(ru_ref, ri_ref, iu_ref, ii_ref, ou_ref, oi_ref):
    su = iu_ref[...] & 1
    ru = ru_ref[...]
    ou_ref[...] = jnp.where(su == 0, ru[:, :NH], ru[:, NH:])
    si = ii_ref[...] & 1
    ri = ri_ref[...]
    oi_ref[...] = jnp.where(si == 0, ri[:, :NH], ri[:, NH:])


RB = 2048


def kernel(idx_user, idx_item, emb_user, emb_item, W_user, b_user, W_item, b_item):
    idx_u = idx_user.astype(jnp.int32)
    idx_i = idx_item.astype(jnp.int32)

    pu = _project(emb_user.T, W_user, b_user, VU)
    pi = _project(emb_item.T, W_item, b_item, VI)

    raw_u, raw_i = _sc_gather(
        idx_u.reshape(NW, NCH, CHUNK), idx_i.reshape(NW, NCH, CHUNK), pu, pi)

    grid = (B // RB,)
    ou, oi = pl.pallas_call(
        _select_kernel,
        grid=grid,
        in_specs=[
            pl.BlockSpec((RB, 128), lambda i: (i, 0)),
            pl.BlockSpec((RB, 128), lambda i: (i, 0)),
            pl.BlockSpec((RB, 1), lambda i: (i, 0)),
            pl.BlockSpec((RB, 1), lambda i: (i, 0)),
        ],
        out_specs=[
            pl.BlockSpec((RB, NH), lambda i: (i, 0)),
            pl.BlockSpec((RB, NH), lambda i: (i, 0)),
        ],
        out_shape=[
            jax.ShapeDtypeStruct((B, NH), jnp.float32),
            jax.ShapeDtypeStruct((B, NH), jnp.float32),
        ],
    )(raw_u, raw_i, idx_u.reshape(B, 1), idx_i.reshape(B, 1))
    return (ou, oi)


# speedup vs baseline: 1.8476x; 1.7673x over previous
import functools

import jax
import jax.numpy as jnp
from jax import lax
from jax.experimental import pallas as pl
from jax.experimental.pallas import tpu as pltpu
from jax.experimental.pallas import tpu_sc as plsc

B = 16384
DU = 32
DI = 64
NH = 64
VU = 1000000
VI = 100000

_info = plsc.get_sparse_core_info()
NC = _info.num_cores
NS = _info.num_subcores
NW = NC * NS
BPW = B // NW
CHUNK = 128
NCH = BPW // CHUNK
L = 16

VB = 16384
VBH = VB // 2
SB = 14
HB = SB - 1

_mesh = plsc.VectorSubcoreMesh(core_axis_name="c", subcore_axis_name="s")


def _proj_kernel(t_ref, wt_ref, b_ref, o_ref):
    x = t_ref[...]
    yt = jnp.dot(wt_ref[...], x, preferred_element_type=jnp.float32)
    yt = jnp.maximum(yt + b_ref[...], 0.0)
    tl = lax.transpose(yt[:, :VBH], (1, 0))
    tr = lax.transpose(yt[:, VBH:], (1, 0))
    o_ref[...] = jnp.concatenate([tl, tr], axis=1)


def _project(table_t, w, b, v):
    d = table_t.shape[0]
    nblk = pl.cdiv(v, VB)
    grid = (nblk,)
    return pl.pallas_call(
        _proj_kernel,
        grid=grid,
        in_specs=[
            pl.BlockSpec((d, VB), lambda i: (0, i)),
            pl.BlockSpec((NH, d), lambda i: (0, 0)),
            pl.BlockSpec((NH, 1), lambda i: (0, 0)),
        ],
        out_specs=pl.BlockSpec((VBH, 2 * NH), lambda i: (i, 0)),
        out_shape=jax.ShapeDtypeStruct((nblk * VBH, 2 * NH), jnp.float32),
        compiler_params=pltpu.CompilerParams(vmem_limit_bytes=56 * 1024 * 1024),
    )(table_t, w.T, b.reshape(NH, 1))


@functools.partial(
    pl.kernel,
    mesh=_mesh,
    out_type=[
        jax.ShapeDtypeStruct((B, 128), jnp.float32),
        jax.ShapeDtypeStruct((B, 128), jnp.float32),
    ],
    scratch_types=[
        pltpu.VMEM((NCH, CHUNK), jnp.int32),
        pltpu.VMEM((NCH, CHUNK), jnp.int32),
        pltpu.VMEM((NCH, CHUNK), jnp.int32),
        pltpu.VMEM((NCH, CHUNK), jnp.int32),
        pltpu.VMEM((2, CHUNK, 128), jnp.float32),
        pltpu.VMEM((2, CHUNK, 128), jnp.float32),
        pltpu.SemaphoreType.DMA,
        pltpu.SemaphoreType.DMA,
        pltpu.SemaphoreType.DMA,
        pltpu.SemaphoreType.DMA,
    ],
)
def _sc_gather(idx_u_hbm, idx_i_hbm, pu_hbm, pi_hbm,
               hu_hbm, hi_hbm,
               idx_u_v, idx_i_v, g_u_v, g_i_v, buf_u, buf_i,
               sem_gu, sem_gi, sem_wu, sem_wi):
    wid = lax.axis_index("s") * NC + lax.axis_index("c")
    base = wid * BPW
    pltpu.sync_copy(idx_u_hbm.at[wid], idx_u_v)
    pltpu.sync_copy(idx_i_hbm.at[wid], idx_i_v)
    for j in range(NCH):
        for k in range(CHUNK // L):
            sl = pl.ds(k * L, L)
            iu = idx_u_v[j, sl]
            ii = idx_i_v[j, sl]
            g_u_v[j, sl] = ((iu >> SB) << HB) + (iu & (VBH - 1))
            g_i_v[j, sl] = ((ii >> SB) << HB) + (ii & (VBH - 1))
    wb_u = [None] * NCH
    wb_i = [None] * NCH
    for j in range(NCH):
        bsel = j % 2
        if j >= 2:
            wb_u[j - 2].wait()
            wb_i[j - 2].wait()
        gu = pltpu.async_copy(pu_hbm.at[g_u_v.at[j]], buf_u.at[bsel], sem_gu)
        gi = pltpu.async_copy(pi_hbm.at[g_i_v.at[j]], buf_i.at[bsel], sem_gi)
        gu.wait()
        gi.wait()
        dst = pl.ds(base + j * CHUNK, CHUNK)
        wb_u[j] = pltpu.async_copy(buf_u.at[bsel], hu_hbm.at[dst], sem_wu)
        wb_i[j] = pltpu.async_copy(buf_i.at[bsel], hi_hbm.at[dst], sem_wi)
    for j in range(NCH - 2, NCH):
        wb_u[j].wait()
        wb_i[j].wait()


def _select_kernel(ru_ref, ri_ref, iu_ref, ii_ref, ou_ref, oi_ref):
    su = (iu_ref[...] >> HB) & 1
    ru = ru_ref[...]
    ou_ref[...] = jnp.where(su == 0, ru[:, :NH], ru[:, NH:])
    si = (ii_ref[...] >> HB) & 1
    ri = ri_ref[...]
    oi_ref[...] = jnp.where(si == 0, ri[:, :NH], ri[:, NH:])


RB = 2048


def kernel(idx_user, idx_item, emb_user, emb_item, W_user, b_user, W_item, b_item):
    idx_u = idx_user.astype(jnp.int32)
    idx_i = idx_item.astype(jnp.int32)

    pu = _project(emb_user.T, W_user, b_user, VU)
    pi = _project(emb_item.T, W_item, b_item, VI)

    raw_u, raw_i = _sc_gather(
        idx_u.reshape(NW, NCH, CHUNK), idx_i.reshape(NW, NCH, CHUNK), pu, pi)

    grid = (B // RB,)
    ou, oi = pl.pallas_call(
        _select_kernel,
        grid=grid,
        in_specs=[
            pl.BlockSpec((RB, 128), lambda i: (i, 0)),
            pl.BlockSpec((RB, 128), lambda i: (i, 0)),
            pl.BlockSpec((RB, 1), lambda i: (i, 0)),
            pl.BlockSpec((RB, 1), lambda i: (i, 0)),
        ],
        out_specs=[
            pl.BlockSpec((RB, NH), lambda i: (i, 0)),
            pl.BlockSpec((RB, NH), lambda i: (i, 0)),
        ],
        out_shape=[
            jax.ShapeDtypeStruct((B, NH), jnp.float32),
            jax.ShapeDtypeStruct((B, NH), jnp.float32),
        ],
    )(raw_u, raw_i, idx_u.reshape(B, 1), idx_i.reshape(B, 1))
    return (ou, oi)


# speedup vs baseline: 2.1955x; 1.1883x over previous
import functools

import jax
import jax.numpy as jnp
from jax import lax
from jax.experimental import pallas as pl
from jax.experimental.pallas import tpu as pltpu
from jax.experimental.pallas import tpu_sc as plsc

B = 16384
DU = 32
DI = 64
NH = 64
VU = 1000000
VI = 100000

_info = plsc.get_sparse_core_info()
NC = _info.num_cores
NS = _info.num_subcores
NW = NC * NS
BPW = B // NW
CHUNK = 128
NCH = BPW // CHUNK
L = 16

VB = 16384
SB = 14
Q = VB // 4

_mesh = plsc.VectorSubcoreMesh(core_axis_name="c", subcore_axis_name="s")


def _rne16(bits):
    return (bits + 0x7FFF + ((bits >> 16) & 1)) >> 16


def _proj_kernel(t_ref, wt_ref, b_ref, o_ref):
    x = t_ref[...]
    yt = jnp.dot(wt_ref[...], x, preferred_element_type=jnp.float32)
    yt = jnp.maximum(yt + b_ref[...], 0.0)
    ybits = lax.bitcast_convert_type(yt, jnp.uint32)
    q0 = _rne16(ybits[:, 0 * Q:1 * Q])
    q1 = _rne16(ybits[:, 1 * Q:2 * Q])
    q2 = _rne16(ybits[:, 2 * Q:3 * Q])
    q3 = _rne16(ybits[:, 3 * Q:4 * Q])
    w01 = lax.bitcast_convert_type((q1 << 16) | q0, jnp.float32)
    w23 = lax.bitcast_convert_type((q3 << 16) | q2, jnp.float32)
    t0 = lax.transpose(w01, (1, 0))
    t1 = lax.transpose(w23, (1, 0))
    o_ref[...] = jnp.concatenate([t0, t1], axis=1)


def _project(table_t, w, b, v):
    d = table_t.shape[0]
    nblk = pl.cdiv(v, VB)
    grid = (nblk,)
    return pl.pallas_call(
        _proj_kernel,
        grid=grid,
        in_specs=[
            pl.BlockSpec((d, VB), lambda i: (0, i)),
            pl.BlockSpec((NH, d), lambda i: (0, 0)),
            pl.BlockSpec((NH, 1), lambda i: (0, 0)),
        ],
        out_specs=pl.BlockSpec((Q, 2 * NH), lambda i: (i, 0)),
        out_shape=jax.ShapeDtypeStruct((nblk * Q, 2 * NH), jnp.float32),
        compiler_params=pltpu.CompilerParams(vmem_limit_bytes=56 * 1024 * 1024),
    )(table_t, w.T, b.reshape(NH, 1))


@functools.partial(
    pl.kernel,
    mesh=_mesh,
    out_type=[
        jax.ShapeDtypeStruct((B, 128), jnp.float32),
        jax.ShapeDtypeStruct((B, 128), jnp.float32),
    ],
    scratch_types=[
        pltpu.VMEM((NCH, CHUNK), jnp.int32),
        pltpu.VMEM((NCH, CHUNK), jnp.int32),
        pltpu.VMEM((NCH, CHUNK), jnp.int32),
        pltpu.VMEM((NCH, CHUNK), jnp.int32),
        pltpu.VMEM((2, CHUNK, 128), jnp.float32),
        pltpu.VMEM((2, CHUNK, 128), jnp.float32),
        pltpu.SemaphoreType.DMA,
        pltpu.SemaphoreType.DMA,
        pltpu.SemaphoreType.DMA,
        pltpu.SemaphoreType.DMA,
    ],
)
def _sc_gather(idx_u_hbm, idx_i_hbm, pu_hbm, pi_hbm,
               hu_hbm, hi_hbm,
               idx_u_v, idx_i_v, g_u_v, g_i_v, buf_u, buf_i,
               sem_gu, sem_gi, sem_wu, sem_wi):
    wid = lax.axis_index("s") * NC + lax.axis_index("c")
    base = wid * BPW
    pltpu.sync_copy(idx_u_hbm.at[wid], idx_u_v)
    pltpu.sync_copy(idx_i_hbm.at[wid], idx_i_v)
    for j in range(NCH):
        for k in range(CHUNK // L):
            sl = pl.ds(k * L, L)
            iu = idx_u_v[j, sl]
            ii = idx_i_v[j, sl]
            g_u_v[j, sl] = ((iu >> SB) << 12) + (iu & (Q - 1))
            g_i_v[j, sl] = ((ii >> SB) << 12) + (ii & (Q - 1))
    wb_u = [None] * NCH
    wb_i = [None] * NCH
    for j in range(NCH):
        bsel = j % 2
        if j >= 2:
            wb_u[j - 2].wait()
            wb_i[j - 2].wait()
        gu = pltpu.async_copy(pu_hbm.at[g_u_v.at[j]], buf_u.at[bsel], sem_gu)
        gi = pltpu.async_copy(pi_hbm.at[g_i_v.at[j]], buf_i.at[bsel], sem_gi)
        gu.wait()
        gi.wait()
        dst = pl.ds(base + j * CHUNK, CHUNK)
        wb_u[j] = pltpu.async_copy(buf_u.at[bsel], hu_hbm.at[dst], sem_wu)
        wb_i[j] = pltpu.async_copy(buf_i.at[bsel], hi_hbm.at[dst], sem_wi)
    for j in range(NCH - 2, NCH):
        wb_u[j].wait()
        wb_i[j].wait()


def _unpack_sel(raw, idx):
    sub = (idx >> 12) & 3
    grp = jnp.where((sub >> 1) == 0, raw[:, :NH], raw[:, NH:])
    bits = lax.bitcast_convert_type(grp, jnp.uint32)
    vb = jnp.where((sub & 1) == 1, bits & jnp.uint32(0xFFFF0000), bits << 16)
    return lax.bitcast_convert_type(vb, jnp.float32)


def _select_kernel(ru_ref, ri_ref, iu_ref, ii_ref, ou_ref, oi_ref):
    ou_ref[...] = _unpack_sel(ru_ref[...], iu_ref[...])
    oi_ref[...] = _unpack_sel(ri_ref[...], ii_ref[...])


RB = 2048


def kernel(idx_user, idx_item, emb_user, emb_item, W_user, b_user, W_item, b_item):
    idx_u = idx_user.astype(jnp.int32)
    idx_i = idx_item.astype(jnp.int32)

    pu = _project(emb_user.T, W_user, b_user, VU)
    pi = _project(emb_item.T, W_item, b_item, VI)

    raw_u, raw_i = _sc_gather(
        idx_u.reshape(NW, NCH, CHUNK), idx_i.reshape(NW, NCH, CHUNK), pu, pi)

    grid = (B // RB,)
    ou, oi = pl.pallas_call(
        _select_kernel,
        grid=grid,
        in_specs=[
            pl.BlockSpec((RB, 128), lambda i: (i, 0)),
            pl.BlockSpec((RB, 128), lambda i: (i, 0)),
            pl.BlockSpec((RB, 1), lambda i: (i, 0)),
            pl.BlockSpec((RB, 1), lambda i: (i, 0)),
        ],
        out_specs=[
            pl.BlockSpec((RB, NH), lambda i: (i, 0)),
            pl.BlockSpec((RB, NH), lambda i: (i, 0)),
        ],
        out_shape=[
            jax.ShapeDtypeStruct((B, NH), jnp.float32),
            jax.ShapeDtypeStruct((B, NH), jnp.float32),
        ],
    )(raw_u, raw_i, idx_u.reshape(B, 1), idx_i.reshape(B, 1))
    return (ou, oi)


# speedup vs baseline: 2.6577x; 1.2106x over previous
import functools

import jax
import jax.numpy as jnp
from jax import lax
from jax.experimental import pallas as pl
from jax.experimental.pallas import tpu as pltpu
from jax.experimental.pallas import tpu_sc as plsc

B = 16384
DU = 32
DI = 64
NH = 64
VU = 1000000
VI = 100000

_info = plsc.get_sparse_core_info()
NC = _info.num_cores
NS = _info.num_subcores
NW = NC * NS
BPW = B // NW
CHUNK = 128
NCH = BPW // CHUNK
L = 16

VB = 16384
SB = 14
Q = VB // 4

_mesh = plsc.VectorSubcoreMesh(core_axis_name="c", subcore_axis_name="s")


def _proj_kernel(t_ref, wt_ref, b_ref, o_ref):
    x = t_ref[...]
    yt = jnp.dot(wt_ref[...], x, preferred_element_type=jnp.float32)
    yt = jnp.maximum(yt + b_ref[...], 0.0)
    ybits = lax.bitcast_convert_type(yt, jnp.uint32)
    q0 = ybits[:, 0 * Q:1 * Q]
    q1 = ybits[:, 1 * Q:2 * Q]
    q2 = ybits[:, 2 * Q:3 * Q]
    q3 = ybits[:, 3 * Q:4 * Q]
    hm = jnp.uint32(0xFFFF0000)
    w01 = lax.bitcast_convert_type((q1 & hm) | (q0 >> 16), jnp.float32)
    w23 = lax.bitcast_convert_type((q3 & hm) | (q2 >> 16), jnp.float32)
    t0 = lax.transpose(w01, (1, 0))
    t1 = lax.transpose(w23, (1, 0))
    o_ref[...] = jnp.concatenate([t0, t1], axis=1)


def _project(table_t, w, b, v):
    d = table_t.shape[0]
    nblk = pl.cdiv(v, VB)
    grid = (nblk,)
    return pl.pallas_call(
        _proj_kernel,
        grid=grid,
        in_specs=[
            pl.BlockSpec((d, VB), lambda i: (0, i)),
            pl.BlockSpec((NH, d), lambda i: (0, 0)),
            pl.BlockSpec((NH, 1), lambda i: (0, 0)),
        ],
        out_specs=pl.BlockSpec((Q, 2 * NH), lambda i: (i, 0)),
        out_shape=jax.ShapeDtypeStruct((nblk * Q, 2 * NH), jnp.float32),
        compiler_params=pltpu.CompilerParams(vmem_limit_bytes=56 * 1024 * 1024),
    )(table_t, w.T, b.reshape(NH, 1))


@functools.partial(
    pl.kernel,
    mesh=_mesh,
    out_type=[
        jax.ShapeDtypeStruct((B, 128), jnp.float32),
        jax.ShapeDtypeStruct((B, 128), jnp.float32),
    ],
    scratch_types=[
        pltpu.VMEM((NCH, CHUNK), jnp.int32),
        pltpu.VMEM((NCH, CHUNK), jnp.int32),
        pltpu.VMEM((NCH, CHUNK), jnp.int32),
        pltpu.VMEM((NCH, CHUNK), jnp.int32),
        pltpu.VMEM((2, CHUNK, 128), jnp.float32),
        pltpu.VMEM((2, CHUNK, 128), jnp.float32),
        pltpu.SemaphoreType.DMA,
        pltpu.SemaphoreType.DMA,
        pltpu.SemaphoreType.DMA,
        pltpu.SemaphoreType.DMA,
    ],
)
def _sc_gather(idx_u_hbm, idx_i_hbm, pu_hbm, pi_hbm,
               hu_hbm, hi_hbm,
               idx_u_v, idx_i_v, g_u_v, g_i_v, buf_u, buf_i,
               sem_gu, sem_gi, sem_wu, sem_wi):
    wid = lax.axis_index("s") * NC + lax.axis_index("c")
    base = wid * BPW
    pltpu.sync_copy(idx_u_hbm.at[wid], idx_u_v)
    pltpu.sync_copy(idx_i_hbm.at[wid], idx_i_v)
    for j in range(NCH):
        for k in range(CHUNK // L):
            sl = pl.ds(k * L, L)
            iu = idx_u_v[j, sl]
            ii = idx_i_v[j, sl]
            g_u_v[j, sl] = ((iu >> SB) << 12) + (iu & (Q - 1))
            g_i_v[j, sl] = ((ii >> SB) << 12) + (ii & (Q - 1))
    wb_u = [None] * NCH
    wb_i = [None] * NCH
    for j in range(NCH):
        bsel = j % 2
        if j >= 2:
            wb_u[j - 2].wait()
            wb_i[j - 2].wait()
        gu = pltpu.async_copy(pu_hbm.at[g_u_v.at[j]], buf_u.at[bsel], sem_gu)
        gi = pltpu.async_copy(pi_hbm.at[g_i_v.at[j]], buf_i.at[bsel], sem_gi)
        gu.wait()
        gi.wait()
        dst = pl.ds(base + j * CHUNK, CHUNK)
        wb_u[j] = pltpu.async_copy(buf_u.at[bsel], hu_hbm.at[dst], sem_wu)
        wb_i[j] = pltpu.async_copy(buf_i.at[bsel], hi_hbm.at[dst], sem_wi)
    for j in range(NCH - 2, NCH):
        wb_u[j].wait()
        wb_i[j].wait()


def _unpack_sel(raw, idx):
    sub = (idx >> 12) & 3
    grp = jnp.where((sub >> 1) == 0, raw[:, :NH], raw[:, NH:])
    bits = lax.bitcast_convert_type(grp, jnp.uint32)
    vb = jnp.where((sub & 1) == 1, bits & jnp.uint32(0xFFFF0000), bits << 16)
    return lax.bitcast_convert_type(vb, jnp.float32)


def _select_kernel(ru_ref, ri_ref, iu_ref, ii_ref, ou_ref, oi_ref):
    ou_ref[...] = _unpack_sel(ru_ref[...], iu_ref[...])
    oi_ref[...] = _unpack_sel(ri_ref[...], ii_ref[...])


RB = 4096


def kernel(idx_user, idx_item, emb_user, emb_item, W_user, b_user, W_item, b_item):
    idx_u = idx_user.astype(jnp.int32)
    idx_i = idx_item.astype(jnp.int32)

    pu = _project(emb_user.T, W_user, b_user, VU)
    pi = _project(emb_item.T, W_item, b_item, VI)

    raw_u, raw_i = _sc_gather(
        idx_u.reshape(NW, NCH, CHUNK), idx_i.reshape(NW, NCH, CHUNK), pu, pi)

    grid = (B // RB,)
    ou, oi = pl.pallas_call(
        _select_kernel,
        grid=grid,
        in_specs=[
            pl.BlockSpec((RB, 128), lambda i: (i, 0)),
            pl.BlockSpec((RB, 128), lambda i: (i, 0)),
            pl.BlockSpec((RB, 1), lambda i: (i, 0)),
            pl.BlockSpec((RB, 1), lambda i: (i, 0)),
        ],
        out_specs=[
            pl.BlockSpec((RB, NH), lambda i: (i, 0)),
            pl.BlockSpec((RB, NH), lambda i: (i, 0)),
        ],
        out_shape=[
            jax.ShapeDtypeStruct((B, NH), jnp.float32),
            jax.ShapeDtypeStruct((B, NH), jnp.float32),
        ],
    )(raw_u, raw_i, idx_u.reshape(B, 1), idx_i.reshape(B, 1))
    return (ou, oi)


# speedup vs baseline: 2.8525x; 1.0733x over previous
import functools

import jax
import jax.numpy as jnp
from jax import lax
from jax.experimental import pallas as pl
from jax.experimental.pallas import tpu as pltpu
from jax.experimental.pallas import tpu_sc as plsc

B = 16384
DU = 32
DI = 64
NH = 64
VU = 1000000
VI = 100000

_info = plsc.get_sparse_core_info()
NC = _info.num_cores
NS = _info.num_subcores
NW = NC * NS
BPW = B // NW
CHUNK = 128
NCH = BPW // CHUNK
L = 16

VB = 32768
SB = 15
Q = VB // 4
QSH = SB - 2

_mesh = plsc.VectorSubcoreMesh(core_axis_name="c", subcore_axis_name="s")


def _proj_kernel(t_ref, wt_ref, b_ref, o_ref):
    x = t_ref[...]
    yt = jnp.dot(wt_ref[...], x, preferred_element_type=jnp.float32)
    yt = jnp.maximum(yt + b_ref[...], 0.0)
    ybits = lax.bitcast_convert_type(yt, jnp.uint32)
    q0 = ybits[:, 0 * Q:1 * Q]
    q1 = ybits[:, 1 * Q:2 * Q]
    q2 = ybits[:, 2 * Q:3 * Q]
    q3 = ybits[:, 3 * Q:4 * Q]
    hm = jnp.uint32(0xFFFF0000)
    w01 = lax.bitcast_convert_type((q1 & hm) | (q0 >> 16), jnp.float32)
    w23 = lax.bitcast_convert_type((q3 & hm) | (q2 >> 16), jnp.float32)
    t0 = lax.transpose(w01, (1, 0))
    t1 = lax.transpose(w23, (1, 0))
    o_ref[...] = jnp.concatenate([t0, t1], axis=1)


def _project(table_t, w, b, v):
    d = table_t.shape[0]
    nblk = pl.cdiv(v, VB)
    grid = (nblk,)
    return pl.pallas_call(
        _proj_kernel,
        grid=grid,
        in_specs=[
            pl.BlockSpec((d, VB), lambda i: (0, i)),
            pl.BlockSpec((NH, d), lambda i: (0, 0)),
            pl.BlockSpec((NH, 1), lambda i: (0, 0)),
        ],
        out_specs=pl.BlockSpec((Q, 2 * NH), lambda i: (i, 0)),
        out_shape=jax.ShapeDtypeStruct((nblk * Q, 2 * NH), jnp.float32),
        compiler_params=pltpu.CompilerParams(vmem_limit_bytes=56 * 1024 * 1024),
    )(table_t, w.T, b.reshape(NH, 1))


@functools.partial(
    pl.kernel,
    mesh=_mesh,
    out_type=[
        jax.ShapeDtypeStruct((B, 128), jnp.float32),
        jax.ShapeDtypeStruct((B, 128), jnp.float32),
    ],
    scratch_types=[
        pltpu.VMEM((NCH, CHUNK), jnp.int32),
        pltpu.VMEM((NCH, CHUNK), jnp.int32),
        pltpu.VMEM((NCH, CHUNK), jnp.int32),
        pltpu.VMEM((NCH, CHUNK), jnp.int32),
        pltpu.VMEM((2, CHUNK, 128), jnp.float32),
        pltpu.VMEM((2, CHUNK, 128), jnp.float32),
        pltpu.SemaphoreType.DMA,
        pltpu.SemaphoreType.DMA,
        pltpu.SemaphoreType.DMA,
        pltpu.SemaphoreType.DMA,
    ],
)
def _sc_gather(idx_u_hbm, idx_i_hbm, pu_hbm, pi_hbm,
               hu_hbm, hi_hbm,
               idx_u_v, idx_i_v, g_u_v, g_i_v, buf_u, buf_i,
               sem_gu, sem_gi, sem_wu, sem_wi):
    wid = lax.axis_index("s") * NC + lax.axis_index("c")
    base = wid * BPW
    pltpu.sync_copy(idx_u_hbm.at[wid], idx_u_v)
    pltpu.sync_copy(idx_i_hbm.at[wid], idx_i_v)
    for j in range(NCH):
        for k in range(CHUNK // L):
            sl = pl.ds(k * L, L)
            iu = idx_u_v[j, sl]
            ii = idx_i_v[j, sl]
            g_u_v[j, sl] = ((iu >> SB) << QSH) + (iu & (Q - 1))
            g_i_v[j, sl] = ((ii >> SB) << QSH) + (ii & (Q - 1))
    wb_u = [None] * NCH
    wb_i = [None] * NCH
    for j in range(NCH):
        bsel = j % 2
        if j >= 2:
            wb_u[j - 2].wait()
            wb_i[j - 2].wait()
        gu = pltpu.async_copy(pu_hbm.at[g_u_v.at[j]], buf_u.at[bsel], sem_gu)
        gi = pltpu.async_copy(pi_hbm.at[g_i_v.at[j]], buf_i.at[bsel], sem_gi)
        gu.wait()
        gi.wait()
        dst = pl.ds(base + j * CHUNK, CHUNK)
        wb_u[j] = pltpu.async_copy(buf_u.at[bsel], hu_hbm.at[dst], sem_wu)
        wb_i[j] = pltpu.async_copy(buf_i.at[bsel], hi_hbm.at[dst], sem_wi)
    for j in range(NCH - 2, NCH):
        wb_u[j].wait()
        wb_i[j].wait()


def _unpack_sel(raw, idx):
    sub = (idx >> QSH) & 3
    grp = jnp.where((sub >> 1) == 0, raw[:, :NH], raw[:, NH:])
    bits = lax.bitcast_convert_type(grp, jnp.uint32)
    vb = jnp.where((sub & 1) == 1, bits & jnp.uint32(0xFFFF0000), bits << 16)
    return lax.bitcast_convert_type(vb, jnp.float32)


def _select_kernel(ru_ref, ri_ref, iu_ref, ii_ref, ou_ref, oi_ref):
    ou_ref[...] = _unpack_sel(ru_ref[...], iu_ref[...])
    oi_ref[...] = _unpack_sel(ri_ref[...], ii_ref[...])


RB = 4096


def kernel(idx_user, idx_item, emb_user, emb_item, W_user, b_user, W_item, b_item):
    idx_u = idx_user.astype(jnp.int32)
    idx_i = idx_item.astype(jnp.int32)

    pu = _project(emb_user.T, W_user, b_user, VU)
    pi = _project(emb_item.T, W_item, b_item, VI)

    raw_u, raw_i = _sc_gather(
        idx_u.reshape(NW, NCH, CHUNK), idx_i.reshape(NW, NCH, CHUNK), pu, pi)

    grid = (B // RB,)
    ou, oi = pl.pallas_call(
        _select_kernel,
        grid=grid,
        in_specs=[
            pl.BlockSpec((RB, 128), lambda i: (i, 0)),
            pl.BlockSpec((RB, 128), lambda i: (i, 0)),
            pl.BlockSpec((RB, 1), lambda i: (i, 0)),
            pl.BlockSpec((RB, 1), lambda i: (i, 0)),
        ],
        out_specs=[
            pl.BlockSpec((RB, NH), lambda i: (i, 0)),
            pl.BlockSpec((RB, NH), lambda i: (i, 0)),
        ],
        out_shape=[
            jax.ShapeDtypeStruct((B, NH), jnp.float32),
            jax.ShapeDtypeStruct((B, NH), jnp.float32),
        ],
    )(raw_u, raw_i, idx_u.reshape(B, 1), idx_i.reshape(B, 1))
    return (ou, oi)
